# baseline (speedup 1.0000x reference)
import functools
import math

import jax
import jax.numpy as jnp
from jax import lax
from jax.experimental import pallas as pl
from jax.experimental.pallas import tpu as pltpu
from jax.experimental.pallas import tpu_sc as plsc

_NUM_CORES = 2
_NUM_SUBCORES = 16
_NW = _NUM_CORES * _NUM_SUBCORES
_CHUNK = 128
_NG = 4


def _transpose_out(gathered, b, s, d_model, scale):
    half = b // 2

    def body(x_ref, o_ref):
        zt = jnp.transpose(x_ref[...]) * scale
        o_ref[0, :, :half] = zt[:d_model]
        o_ref[0, :, half:] = zt[d_model:]

    return pl.pallas_call(
        body,
        grid=(s,),
        in_specs=[pl.BlockSpec((half, 2 * d_model), lambda i: (i, 0))],
        out_specs=pl.BlockSpec((1, d_model, b), lambda i: (i, 0, 0)),
        out_shape=jax.ShapeDtypeStruct((s, d_model, b), jnp.float32),
    )(gathered)


def _make_lookup(n_idx, vocab, d_model):
    per_w = n_idx // _NW
    n_chunk = per_w // _CHUNK
    mesh = plsc.VectorSubcoreMesh(core_axis_name="c", subcore_axis_name="s")

    @functools.partial(
        pl.kernel,
        mesh=mesh,
        out_type=jax.ShapeDtypeStruct((n_idx, d_model), jnp.float32),
        scratch_types=[
            pltpu.VMEM((per_w,), jnp.int32),
            pltpu.VMEM((_NG, _CHUNK, d_model), jnp.float32),
            pltpu.SemaphoreType.DMA,
            pltpu.SemaphoreType.DMA,
        ],
        compiler_params=pltpu.CompilerParams(use_tc_tiling_on_sc=False),
    )
    def lookup(table_hbm, idx_hbm, out_hbm, idx_v, gbuf, gsem, ssem):
        n_group = n_chunk // _NG
        wid = lax.axis_index("s") * _NUM_CORES + lax.axis_index("c")
        pltpu.sync_copy(idx_hbm.at[pl.ds(wid * per_w, per_w)], idx_v)

        def gather_desc(j, b):
            return pltpu.make_async_copy(
                table_hbm.at[idx_v.at[pl.ds(j * _CHUNK, _CHUNK)]],
                gbuf.at[b],
                gsem,
            )

        def scatter_desc(j, b):
            base = (wid * n_chunk + j) * _CHUNK
            return pltpu.make_async_copy(
                gbuf.at[b], out_hbm.at[pl.ds(base, _CHUNK)], ssem
            )

        def group(g, carry):
            j0 = g * _NG

            @pl.when(g > 0)
            def _():
                for b in range(_NG):
                    scatter_desc(j0 - _NG + b, b).wait()

            for b in range(_NG):
                gather_desc(j0 + b, b).start()
            for b in range(_NG):
                gather_desc(j0 + b, b).wait()
            for b in range(_NG):
                scatter_desc(j0 + b, b).start()
            return carry

        lax.fori_loop(0, n_group, group, 0)
        for b in range(_NG):
            scatter_desc(n_chunk - _NG + b, b).wait()

    return lookup


def kernel(X, table):
    b, s = X.shape
    vocab, d_model = table.shape
    n_idx = b * s
    half = b // 2
    scale = math.sqrt(d_model)
    idx = (
        jnp.transpose(X)
        .reshape(s, 2, half)
        .transpose(0, 2, 1)
        .reshape(n_idx)
        .astype(jnp.int32)
    )
    gathered = _make_lookup(n_idx, vocab, d_model)(table, idx)
    packed = gathered.reshape(n_idx // 2, 2 * d_model)
    out = _transpose_out(packed, b, s, d_model, scale)
    return jnp.transpose(out, (2, 0, 1))


# speedup vs baseline: 1.4288x; 1.4288x over previous
import functools
import math

import jax
import jax.numpy as jnp
from jax import lax
from jax.experimental import pallas as pl
from jax.experimental.pallas import tpu as pltpu
from jax.experimental.pallas import tpu_sc as plsc

_NUM_CORES = 2
_NUM_SUBCORES = 16
_NW = _NUM_CORES * _NUM_SUBCORES
_CHUNK = 128
_NG = 4
_CONV_BLK = 4096


def _conv_table(table_t):
    d_model, vocab = table_t.shape
    grid = (vocab + _CONV_BLK - 1) // _CONV_BLK

    def body(x_ref, o_ref):
        o_ref[:, :d_model] = jnp.transpose(x_ref[...])

    return pl.pallas_call(
        body,
        grid=(grid,),
        in_specs=[pl.BlockSpec((d_model, _CONV_BLK), lambda i: (0, i))],
        out_specs=pl.BlockSpec((_CONV_BLK, 128), lambda i: (i, 0)),
        out_shape=jax.ShapeDtypeStruct((vocab, 128), jnp.float32),
    )(table_t)


def _transpose_out(packed, b, s, d_model, scale):
    half = b // 2

    def body(x_ref, o_ref):
        zt = jnp.transpose(x_ref[...]) * scale
        o_ref[0, :, :half] = zt[:d_model]
        o_ref[0, :, half:] = zt[d_model:]

    return pl.pallas_call(
        body,
        grid=(s,),
        in_specs=[pl.BlockSpec((half, 2 * d_model), lambda i: (i, 0))],
        out_specs=pl.BlockSpec((1, d_model, b), lambda i: (i, 0, 0)),
        out_shape=jax.ShapeDtypeStruct((s, d_model, b), jnp.float32),
    )(packed)


def _make_lookup(n_idx, vocab, d_model):
    per_w = n_idx // _NW
    n_chunk = per_w // _CHUNK
    mesh = plsc.VectorSubcoreMesh(core_axis_name="c", subcore_axis_name="s")

    @functools.partial(
        pl.kernel,
        mesh=mesh,
        out_type=jax.ShapeDtypeStruct((n_idx, d_model), jnp.float32),
        scratch_types=[
            pltpu.VMEM((per_w,), jnp.int32),
            pltpu.VMEM((_NG, _CHUNK, d_model), jnp.float32),
            pltpu.SemaphoreType.DMA,
            pltpu.SemaphoreType.DMA,
        ],
        compiler_params=pltpu.CompilerParams(use_tc_tiling_on_sc=False),
    )
    def lookup(table_hbm, idx_hbm, out_hbm, idx_v, gbuf, gsem, ssem):
        n_group = n_chunk // _NG
        wid = lax.axis_index("s") * _NUM_CORES + lax.axis_index("c")
        pltpu.sync_copy(idx_hbm.at[pl.ds(wid * per_w, per_w)], idx_v)

        def gather_desc(j, b):
            return pltpu.make_async_copy(
                table_hbm.at[idx_v.at[pl.ds(j * _CHUNK, _CHUNK)]],
                gbuf.at[b],
                gsem,
            )

        def scatter_desc(j, b):
            base = (wid * n_chunk + j) * _CHUNK
            return pltpu.make_async_copy(
                gbuf.at[b], out_hbm.at[pl.ds(base, _CHUNK)], ssem
            )

        def group(g, carry):
            j0 = g * _NG

            @pl.when(g > 0)
            def _():
                for b in range(_NG):
                    scatter_desc(j0 - _NG + b, b).wait()

            for b in range(_NG):
                gather_desc(j0 + b, b).start()
            for b in range(_NG):
                gather_desc(j0 + b, b).wait()
            for b in range(_NG):
                scatter_desc(j0 + b, b).start()
            return carry

        lax.fori_loop(0, n_group, group, 0)
        for b in range(_NG):
            scatter_desc(n_chunk - _NG + b, b).wait()

    return lookup


def kernel(X, table):
    b, s = X.shape
    vocab, d_model = table.shape
    n_idx = b * s
    scale = math.sqrt(d_model)
    pad_w = 128 // d_model
    table_c = _conv_table(jnp.transpose(table)).reshape(
        pad_w * vocab, d_model
    )
    half = b // 2
    flat = jnp.transpose(X).reshape(n_idx).astype(jnp.int32)
    k = jnp.arange(n_idx, dtype=jnp.int32)
    perm = (k // b) * b + (k % 2) * half + (k % b) // 2
    idx = pad_w * jnp.take(flat, perm)
    gathered = _make_lookup(n_idx, pad_w * vocab, d_model)(table_c, idx)
    packed = gathered.reshape(n_idx // 2, 2 * d_model)
    out = _transpose_out(packed, b, s, d_model, scale)
    return jnp.transpose(out, (2, 0, 1))


# speedup vs baseline: 1.4732x; 1.0311x over previous
import functools
import math

import jax
import jax.numpy as jnp
from jax import lax
from jax.experimental import pallas as pl
from jax.experimental.pallas import tpu as pltpu
from jax.experimental.pallas import tpu_sc as plsc

_NUM_CORES = 2
_NUM_SUBCORES = 16
_NW = _NUM_CORES * _NUM_SUBCORES
_CHUNK = 128
_NG = 4
_CONV_BLK = 4096


def _conv_table(table_t):
    d_model, vocab = table_t.shape
    grid = (vocab + _CONV_BLK - 1) // _CONV_BLK

    def body(x_ref, o_ref):
        o_ref[:, :d_model] = jnp.transpose(x_ref[...])

    return pl.pallas_call(
        body,
        grid=(grid,),
        in_specs=[pl.BlockSpec((d_model, _CONV_BLK), lambda i: (0, i))],
        out_specs=pl.BlockSpec((_CONV_BLK, 128), lambda i: (i, 0)),
        out_shape=jax.ShapeDtypeStruct((vocab, 128), jnp.float32),
    )(table_t)


def _transpose_out(packed, b, s, d_model, scale):
    half = b // 2

    def body(x_ref, o_ref):
        zt = jnp.transpose(x_ref[...]) * scale
        o_ref[0, :, :half] = zt[:d_model]
        o_ref[0, :, half:] = zt[d_model:]

    return pl.pallas_call(
        body,
        grid=(s,),
        in_specs=[pl.BlockSpec((half, 2 * d_model), lambda i: (i, 0))],
        out_specs=pl.BlockSpec((1, d_model, b), lambda i: (i, 0, 0)),
        out_shape=jax.ShapeDtypeStruct((s, d_model, b), jnp.float32),
    )(packed)


def _make_lookup(n_idx, vocab, d_model):
    per_w = n_idx // _NW
    n_chunk = per_w // _CHUNK
    mesh = plsc.VectorSubcoreMesh(core_axis_name="c", subcore_axis_name="s")

    @functools.partial(
        pl.kernel,
        mesh=mesh,
        out_type=jax.ShapeDtypeStruct((n_idx, d_model), jnp.float32),
        scratch_types=[
            pltpu.VMEM((per_w,), jnp.int32),
            pltpu.VMEM((2 * _NG, _CHUNK, d_model), jnp.float32),
            pltpu.SemaphoreType.DMA,
            pltpu.SemaphoreType.DMA,
            pltpu.SemaphoreType.DMA,
            pltpu.SemaphoreType.DMA,
        ],
        compiler_params=pltpu.CompilerParams(use_tc_tiling_on_sc=False),
    )
    def lookup(table_hbm, idx_hbm, out_hbm, idx_v, gbuf, *sems):
        n_pair = n_chunk // (2 * _NG)
        wid = lax.axis_index("s") * _NUM_CORES + lax.axis_index("c")
        pltpu.sync_copy(idx_hbm.at[pl.ds(wid * per_w, per_w)], idx_v)

        def gather_desc(j, b, sem):
            return pltpu.make_async_copy(
                table_hbm.at[idx_v.at[pl.ds(j * _CHUNK, _CHUNK)]],
                gbuf.at[b],
                sem,
            )

        def scatter_desc(j, b, sem):
            base = (wid * n_chunk + j) * _CHUNK
            return pltpu.make_async_copy(
                gbuf.at[b], out_hbm.at[pl.ds(base, _CHUNK)], sem
            )

        def pair(gg, carry):
            j0 = 2 * gg * _NG

            @pl.when(gg > 0)
            def _():
                for b in range(_NG):
                    scatter_desc(j0 - 2 * _NG + b, b, sems[2]).wait()

            for b in range(_NG):
                gather_desc(j0 + b, b, sems[0]).start()

            @pl.when(gg > 0)
            def _():
                for b in range(_NG):
                    scatter_desc(j0 - _NG + b, _NG + b, sems[3]).wait()

            for b in range(_NG):
                gather_desc(j0 + _NG + b, _NG + b, sems[1]).start()
            for b in range(_NG):
                gather_desc(j0 + b, b, sems[0]).wait()
            for b in range(_NG):
                scatter_desc(j0 + b, b, sems[2]).start()
            for b in range(_NG):
                gather_desc(j0 + _NG + b, _NG + b, sems[1]).wait()
            for b in range(_NG):
                scatter_desc(j0 + _NG + b, _NG + b, sems[3]).start()
            return carry

        lax.fori_loop(0, n_pair, pair, 0)
        for b in range(_NG):
            scatter_desc(n_chunk - 2 * _NG + b, b, sems[2]).wait()
        for b in range(_NG):
            scatter_desc(n_chunk - _NG + b, _NG + b, sems[3]).wait()

    return lookup


def kernel(X, table):
    b, s = X.shape
    vocab, d_model = table.shape
    n_idx = b * s
    scale = math.sqrt(d_model)
    pad_w = 128 // d_model
    table_c = _conv_table(jnp.transpose(table)).reshape(
        pad_w * vocab, d_model
    )
    half = b // 2
    flat = jnp.transpose(X).reshape(n_idx).astype(jnp.int32)
    k = jnp.arange(n_idx, dtype=jnp.int32)
    perm = (k // b) * b + (k % 2) * half + (k % b) // 2
    idx = pad_w * jnp.take(flat, perm)
    gathered = _make_lookup(n_idx, pad_w * vocab, d_model)(table_c, idx)
    packed = gathered.reshape(n_idx // 2, 2 * d_model)
    out = _transpose_out(packed, b, s, d_model, scale)
    return jnp.transpose(out, (2, 0, 1))
